# baseline (speedup 1.0000x reference)
import functools

import jax
import jax.numpy as jnp
from jax import lax
from jax.experimental import pallas as pl
from jax.experimental.pallas import tpu as pltpu
from jax.experimental.pallas import tpu_sc as plsc

N = 10000
E = 320000
D = 128

NC = 2
NS = 16
NW = NC * NS

EPT = E // NW
KBLK = 40
NBLK = EPT // KBLK
SUP = 50
NSUP = NBLK // SUP
NQ = SUP // 4 - 1

NP = 10240
RPT = NP // NS
NROWS = 4 * KBLK

UNROLL = 8

_MESH = plsc.VectorSubcoreMesh(core_axis_name="c", subcore_axis_name="s")


@functools.partial(
    pl.kernel,
    mesh=_MESH,
    out_type=jax.ShapeDtypeStruct((NC, NP, D), jnp.float32),
    scratch_types=[
        pltpu.VMEM((SUP, KBLK), jnp.int32),
        pltpu.VMEM((SUP, KBLK), jnp.int32),
        pltpu.VMEM((SUP, KBLK), jnp.float32),
        pltpu.VMEM((NROWS, D), jnp.float32),
        pltpu.VMEM_SHARED((NP, D), jnp.float32),
        pltpu.SemaphoreType.DMA((4,)),
        pltpu.SemaphoreType.DMA((4,)),
    ],
    compiler_params=pltpu.CompilerParams(needs_layout_passes=False),
)
def _sc_scatter(feat, src4, dst4, ev4, out, src_i, dst_i, ev_v, rows,
                acc, sem_g, sem_s):
    c = lax.axis_index("c")
    s = lax.axis_index("s")
    wid = s * NC + c

    @plsc.parallel_loop(0, NROWS * 8, unroll=8)
    def _zero(i):
        r = i // 8
        j = i % 8
        rows[r, pl.ds(j * 16, 16)] = jnp.zeros((16,), jnp.float32)

    for i in range(RPT // NROWS):
        pltpu.sync_copy(rows.at[pl.ds(0, NROWS)],
                        acc.at[pl.ds(s * RPT + i * NROWS, NROWS)])

    plsc.subcore_barrier()

    def _scale(p, slot, b):
        rvec = jnp.full((16,), slot, jnp.int32)

        @plsc.parallel_loop(0, KBLK, unroll=UNROLL)
        def _sc_u(k):
            evb = plsc.load_gather(
                ev_v, [rvec, jnp.full((16,), k, jnp.int32)])
            r = p * KBLK + k
            for j in range(8):
                sl = pl.ds(j * 16, 16)
                rows[r, sl] = rows[r, sl] * evb

    def _gather_start(slot, p, b):
        pltpu.async_copy(feat.at[src_i.at[slot]],
                         rows.at[pl.ds(p * KBLK, KBLK)], sem_g.at[p])

    def _gather_wait(slot, p, b):
        pltpu.make_async_copy(
            feat.at[src_i.at[slot]],
            rows.at[pl.ds(p * KBLK, KBLK)], sem_g.at[p]).wait()

    def _scatter_start(slot, p, b):
        pltpu.async_copy(rows.at[pl.ds(p * KBLK, KBLK)],
                         acc.at[dst_i.at[slot]], sem_s.at[p],
                         add=True)

    def _scatter_wait(slot, p, b):
        pltpu.make_async_copy(
            rows.at[pl.ds(p * KBLK, KBLK)],
            acc.at[dst_i.at[slot]], sem_s.at[p]).wait()

    def _stage_mid(slot, i, b):
        _scatter_wait(slot - 2, (i + 2) % 4, b)
        _gather_start(slot + 2, (i + 2) % 4, b)
        _gather_wait(slot, i % 4, b)
        _scale(i % 4, slot, b)
        _scatter_start(slot, i % 4, b)

    def _sup(sup, carry):
        b = 0
        pltpu.sync_copy(src4.at[wid, sup], src_i)
        pltpu.sync_copy(dst4.at[wid, sup], dst_i)
        pltpu.sync_copy(ev4.at[wid, sup], ev_v)

        _gather_start(0, 0, b)
        _gather_start(1, 1, b)

        for i in range(2):
            _gather_start(i + 2, i + 2, b)
            _gather_wait(i, i, b)
            _scale(i, i, b)
            _scatter_start(i, i, b)
        for i in range(2, 4):
            _scatter_wait(i - 2, (i + 2) % 4, b)
            _gather_start(i + 2, (i + 2) % 4, b)
            _gather_wait(i, i, b)
            _scale(i, i, b)
            _scatter_start(i, i, b)

        def _quad(q, carry2):
            base = 4 * (q + 1)
            for i in range(4):
                _stage_mid(base + i, i, b)
            return carry2

        lax.fori_loop(0, NQ, _quad, 0)

        for off in range(2):
            slot = SUP - 2 + off
            i = slot % 4
            _scatter_wait(slot - 2, (i + 2) % 4, b)
            _gather_wait(slot, i, b)
            _scale(i, slot, b)
            _scatter_start(slot, i, b)
        _scatter_wait(SUP - 2, (SUP - 2) % 4, b)
        _scatter_wait(SUP - 1, (SUP - 1) % 4, b)
        return carry

    lax.fori_loop(0, NSUP, _sup, 0)
    plsc.subcore_barrier()

    pltpu.sync_copy(acc.at[pl.ds(s * RPT, RPT)],
                    out.at[c, pl.ds(s * RPT, RPT)])


RB = 1000


def _tc1_body(x_ref, w_ref, b_ref, sk_ref, o_ref):
    ws = w_ref[...] * sk_ref[...]
    o_ref[...] = (
        jnp.dot(x_ref[...], ws, preferred_element_type=jnp.float32)
        + b_ref[...])


def _tc1(x, w, bias2, skip2):
    return pl.pallas_call(
        _tc1_body,
        grid=(N // RB,),
        in_specs=[
            pl.BlockSpec((RB, D), lambda i: (i, 0)),
            pl.BlockSpec((D, D), lambda i: (0, 0)),
            pl.BlockSpec((1, D), lambda i: (0, 0)),
            pl.BlockSpec((1, D), lambda i: (0, 0)),
        ],
        out_specs=pl.BlockSpec((RB, D), lambda i: (i, 0)),
        out_shape=jax.ShapeDtypeStruct((N, D), jnp.float32),
    )(x, w, bias2, skip2)


def _tc2_body(z_ref, p0_ref, p1_ref, w_ref, o_ref):
    agg = p0_ref[...] + p1_ref[...]
    r = z_ref[...] + jnp.dot(agg, w_ref[...],
                             preferred_element_type=jnp.float32)
    alpha = 1.6732632423543772848170429916717
    scale = 1.0507009873554804934193349852946
    neg = alpha * (jnp.exp(jnp.minimum(r, 0.0)) - 1.0)
    o_ref[...] = scale * jnp.where(r > 0, r, neg)


def _tc2(z, p0, p1, w):
    return pl.pallas_call(
        _tc2_body,
        grid=(N // RB,),
        in_specs=[
            pl.BlockSpec((RB, D), lambda i: (i, 0)),
            pl.BlockSpec((RB, D), lambda i: (i, 0)),
            pl.BlockSpec((RB, D), lambda i: (i, 0)),
            pl.BlockSpec((D, D), lambda i: (0, 0)),
        ],
        out_specs=pl.BlockSpec((RB, D), lambda i: (i, 0)),
        out_shape=jax.ShapeDtypeStruct((N, D), jnp.float32),
    )(z, p0, p1, w)


def kernel(features, edge_index, edge_vals, kernel, bias, skip_weight):
    src4 = edge_index[0].reshape(NW, NSUP, SUP, KBLK)
    dst4 = edge_index[1].reshape(NW, NSUP, SUP, KBLK)
    ev4 = edge_vals.reshape(NW, NSUP, SUP, KBLK)
    partial = _sc_scatter(features, src4, dst4, ev4)
    bias2 = bias.reshape(1, D)
    skip2 = skip_weight.reshape(1, D)
    z = _tc1(features, kernel, bias2, skip2)
    return _tc2(z, partial[0], partial[1], kernel)


# speedup vs baseline: 1.0087x; 1.0087x over previous
import functools

import jax
import jax.numpy as jnp
from jax import lax
from jax.experimental import pallas as pl
from jax.experimental.pallas import tpu as pltpu
from jax.experimental.pallas import tpu_sc as plsc

N = 10000
E = 320000
D = 128

NC = 2
NS = 16
NW = NC * NS

EPT = E // NW
KBLK = 40
NBLK = EPT // KBLK
SUP = 50
NSUP = NBLK // SUP
NQ = SUP // 4 - 1

NP = 10240
RPT = NP // NS
NROWS = 4 * KBLK

UNROLL = 4

_MESH = plsc.VectorSubcoreMesh(core_axis_name="c", subcore_axis_name="s")


@functools.partial(
    pl.kernel,
    mesh=_MESH,
    out_type=jax.ShapeDtypeStruct((NC, NP, D), jnp.float32),
    scratch_types=[
        pltpu.VMEM((SUP, KBLK), jnp.int32),
        pltpu.VMEM((SUP, KBLK), jnp.int32),
        pltpu.VMEM((SUP, KBLK), jnp.float32),
        pltpu.VMEM((NROWS, D), jnp.float32),
        pltpu.VMEM_SHARED((NP, D), jnp.float32),
        pltpu.SemaphoreType.DMA((4,)),
        pltpu.SemaphoreType.DMA((4,)),
    ],
    compiler_params=pltpu.CompilerParams(needs_layout_passes=False),
)
def _sc_scatter(feat, src4, dst4, ev4, out, src_i, dst_i, ev_v, rows,
                acc, sem_g, sem_s):
    c = lax.axis_index("c")
    s = lax.axis_index("s")
    wid = s * NC + c

    @plsc.parallel_loop(0, NROWS * 8, unroll=8)
    def _zero(i):
        r = i // 8
        j = i % 8
        rows[r, pl.ds(j * 16, 16)] = jnp.zeros((16,), jnp.float32)

    for i in range(RPT // NROWS):
        pltpu.sync_copy(rows.at[pl.ds(0, NROWS)],
                        acc.at[pl.ds(s * RPT + i * NROWS, NROWS)])

    plsc.subcore_barrier()

    def _scale(p, slot, b):
        rvec = jnp.full((16,), slot, jnp.int32)

        @plsc.parallel_loop(0, KBLK, unroll=UNROLL)
        def _sc_u(k):
            evb = plsc.load_gather(
                ev_v, [rvec, jnp.full((16,), k, jnp.int32)])
            r = p * KBLK + k
            vals = [rows[r, pl.ds(j * 16, 16)] for j in range(8)]
            scaled = [v * evb for v in vals]
            for j in range(8):
                rows[r, pl.ds(j * 16, 16)] = scaled[j]

    def _gather_start(slot, p, b):
        pltpu.async_copy(feat.at[src_i.at[slot]],
                         rows.at[pl.ds(p * KBLK, KBLK)], sem_g.at[p])

    def _gather_wait(slot, p, b):
        pltpu.make_async_copy(
            feat.at[src_i.at[slot]],
            rows.at[pl.ds(p * KBLK, KBLK)], sem_g.at[p]).wait()

    def _scatter_start(slot, p, b):
        pltpu.async_copy(rows.at[pl.ds(p * KBLK, KBLK)],
                         acc.at[dst_i.at[slot]], sem_s.at[p],
                         add=True)

    def _scatter_wait(slot, p, b):
        pltpu.make_async_copy(
            rows.at[pl.ds(p * KBLK, KBLK)],
            acc.at[dst_i.at[slot]], sem_s.at[p]).wait()

    def _stage_mid(slot, i, b):
        _scatter_wait(slot - 2, (i + 2) % 4, b)
        _gather_start(slot + 2, (i + 2) % 4, b)
        _gather_wait(slot, i % 4, b)
        _scale(i % 4, slot, b)
        _scatter_start(slot, i % 4, b)

    def _sup(sup, carry):
        b = 0
        pltpu.sync_copy(src4.at[wid, sup], src_i)
        pltpu.sync_copy(dst4.at[wid, sup], dst_i)
        pltpu.sync_copy(ev4.at[wid, sup], ev_v)

        _gather_start(0, 0, b)
        _gather_start(1, 1, b)

        for i in range(2):
            _gather_start(i + 2, i + 2, b)
            _gather_wait(i, i, b)
            _scale(i, i, b)
            _scatter_start(i, i, b)
        for i in range(2, 4):
            _scatter_wait(i - 2, (i + 2) % 4, b)
            _gather_start(i + 2, (i + 2) % 4, b)
            _gather_wait(i, i, b)
            _scale(i, i, b)
            _scatter_start(i, i, b)

        def _quad(q, carry2):
            base = 4 * (q + 1)
            for i in range(4):
                _stage_mid(base + i, i, b)
            return carry2

        lax.fori_loop(0, NQ, _quad, 0)

        for off in range(2):
            slot = SUP - 2 + off
            i = slot % 4
            _scatter_wait(slot - 2, (i + 2) % 4, b)
            _gather_wait(slot, i, b)
            _scale(i, slot, b)
            _scatter_start(slot, i, b)
        _scatter_wait(SUP - 2, (SUP - 2) % 4, b)
        _scatter_wait(SUP - 1, (SUP - 1) % 4, b)
        return carry

    lax.fori_loop(0, NSUP, _sup, 0)
    plsc.subcore_barrier()

    pltpu.sync_copy(acc.at[pl.ds(s * RPT, RPT)],
                    out.at[c, pl.ds(s * RPT, RPT)])


RB = 1000


def _tc1_body(x_ref, w_ref, b_ref, sk_ref, o_ref):
    ws = w_ref[...] * sk_ref[...]
    o_ref[...] = (
        jnp.dot(x_ref[...], ws, preferred_element_type=jnp.float32)
        + b_ref[...])


def _tc1(x, w, bias2, skip2):
    return pl.pallas_call(
        _tc1_body,
        grid=(N // RB,),
        in_specs=[
            pl.BlockSpec((RB, D), lambda i: (i, 0)),
            pl.BlockSpec((D, D), lambda i: (0, 0)),
            pl.BlockSpec((1, D), lambda i: (0, 0)),
            pl.BlockSpec((1, D), lambda i: (0, 0)),
        ],
        out_specs=pl.BlockSpec((RB, D), lambda i: (i, 0)),
        out_shape=jax.ShapeDtypeStruct((N, D), jnp.float32),
    )(x, w, bias2, skip2)


def _tc2_body(z_ref, p0_ref, p1_ref, w_ref, o_ref):
    agg = p0_ref[...] + p1_ref[...]
    r = z_ref[...] + jnp.dot(agg, w_ref[...],
                             preferred_element_type=jnp.float32)
    alpha = 1.6732632423543772848170429916717
    scale = 1.0507009873554804934193349852946
    neg = alpha * (jnp.exp(jnp.minimum(r, 0.0)) - 1.0)
    o_ref[...] = scale * jnp.where(r > 0, r, neg)


def _tc2(z, p0, p1, w):
    return pl.pallas_call(
        _tc2_body,
        grid=(N // RB,),
        in_specs=[
            pl.BlockSpec((RB, D), lambda i: (i, 0)),
            pl.BlockSpec((RB, D), lambda i: (i, 0)),
            pl.BlockSpec((RB, D), lambda i: (i, 0)),
            pl.BlockSpec((D, D), lambda i: (0, 0)),
        ],
        out_specs=pl.BlockSpec((RB, D), lambda i: (i, 0)),
        out_shape=jax.ShapeDtypeStruct((N, D), jnp.float32),
    )(z, p0, p1, w)


def kernel(features, edge_index, edge_vals, kernel, bias, skip_weight):
    src4 = edge_index[0].reshape(NW, NSUP, SUP, KBLK)
    dst4 = edge_index[1].reshape(NW, NSUP, SUP, KBLK)
    ev4 = edge_vals.reshape(NW, NSUP, SUP, KBLK)
    partial = _sc_scatter(features, src4, dst4, ev4)
    bias2 = bias.reshape(1, D)
    skip2 = skip_weight.reshape(1, D)
    z = _tc1(features, kernel, bias2, skip2)
    return _tc2(z, partial[0], partial[1], kernel)
